# Initial kernel scaffold; baseline (speedup 1.0000x reference)
import jax
import jax.numpy as jnp
from jax.experimental import pallas as pl


def kernel(x, gate_W, gate_b, expert_W, expert_b, gamma, beta):
    raise NotImplementedError("write your pallas kernel here")


# speedup vs baseline: 2.0851x; 2.0851x over previous
import functools

import jax
import jax.numpy as jnp
from jax.experimental import pallas as pl
from jax.experimental.pallas import tpu as pltpu

INPUT_DIM = 1024
OUTPUT_DIM = 1024
NUM_EXPERTS = 8
EPS = 1e-6
TOKENS = 4096

BT = 512


def _gate_body(x_ref, gw_ref, gb_ref, gamma_ref, beta_ref, w_ref):
    logits = jax.lax.dot_general(
        x_ref[...], gw_ref[...],
        (((1,), (1,)), ((), ())),
        preferred_element_type=jnp.float32,
    ) + gb_ref[...][None, :]
    gx = jnp.sqrt(jnp.sum(logits * logits, axis=1, keepdims=True))
    nx = gx / (jnp.mean(gx, axis=0, keepdims=True) + EPS)
    logits = gamma_ref[...] * (logits * nx) + beta_ref[...]
    m = jnp.max(logits, axis=1, keepdims=True)
    p = jnp.exp(logits - m)
    p = p / jnp.sum(p, axis=1, keepdims=True)
    ii = jax.lax.broadcasted_iota(jnp.int32, p.shape, 1)
    m1 = jnp.max(p, axis=1, keepdims=True)
    i1 = jnp.min(jnp.where(p == m1, ii, NUM_EXPERTS), axis=1, keepdims=True)
    sel1 = ii == i1
    p2 = jnp.where(sel1, -jnp.inf, p)
    m2 = jnp.max(p2, axis=1, keepdims=True)
    i2 = jnp.min(jnp.where(p2 == m2, ii, NUM_EXPERTS), axis=1, keepdims=True)
    w = jnp.where(sel1, m1, 0.0) + jnp.where(ii == i2, m2, 0.0)
    w_ref[...] = w


def _expert_body(w_ref, x_ref, wt_ref, b_ref, out_ref):
    acc = jnp.zeros((BT, OUTPUT_DIM), jnp.float32)
    for e in range(NUM_EXPERTS):
        y = jax.lax.dot_general(
            x_ref[...], wt_ref[e],
            (((1,), (1,)), ((), ())),
            preferred_element_type=jnp.float32,
        )
        y = y + b_ref[e][None, :]
        acc = acc + w_ref[:, e][:, None] * y
    out_ref[...] = acc


@jax.jit
def kernel(x, gate_W, gate_b, expert_W, expert_b, gamma, beta):
    w = pl.pallas_call(
        _gate_body,
        out_shape=jax.ShapeDtypeStruct((TOKENS, NUM_EXPERTS), jnp.float32),
    )(x, gate_W, gate_b, gamma, beta)

    x_bf = x.astype(jnp.bfloat16)
    wt = expert_W.reshape(NUM_EXPERTS, OUTPUT_DIM, INPUT_DIM).astype(jnp.bfloat16)
    b = expert_b.reshape(NUM_EXPERTS, OUTPUT_DIM)

    grid = (TOKENS // BT,)
    out = pl.pallas_call(
        _expert_body,
        grid=grid,
        in_specs=[
            pl.BlockSpec((BT, NUM_EXPERTS), lambda t: (t, 0)),
            pl.BlockSpec((BT, INPUT_DIM), lambda t: (t, 0)),
            pl.BlockSpec((NUM_EXPERTS, OUTPUT_DIM, INPUT_DIM), lambda t: (0, 0, 0)),
            pl.BlockSpec((NUM_EXPERTS, OUTPUT_DIM), lambda t: (0, 0)),
        ],
        out_specs=pl.BlockSpec((BT, OUTPUT_DIM), lambda t: (t, 0)),
        out_shape=jax.ShapeDtypeStruct((TOKENS, OUTPUT_DIM), jnp.float32),
    )(w, x_bf, wt, b)
    return out
